# baseline (speedup 1.0000x reference)
import jax
import jax.numpy as jnp
from jax import lax
from jax.experimental import pallas as pl
from jax.experimental.pallas import tpu as pltpu
from jax.experimental.pallas import tpu_sc as plsc

VOCAB = 100000
EMB = 64
BATCH = 1024
CTX = 20

NC = 2
NS = 16
L = 16
NW = NC * NS
B_PER_W = BATCH // NW
IDX_PER_W = B_PER_W * CTX
GCHUNK = 128
NCHUNK = IDX_PER_W // GCHUNK


def _pool_body(table_hbm, idx_hbm, out_hbm, idx_v, rows_v, pooled_v, sem):
    wid = lax.axis_index("s") * NC + lax.axis_index("c")
    pltpu.sync_copy(idx_hbm.at[wid], idx_v)
    descs = [
        pltpu.async_copy(
            table_hbm.at[idx_v.at[j]],
            rows_v.at[pl.ds(j * GCHUNK, GCHUNK)],
            sem,
        )
        for j in range(NCHUNK)
    ]
    for d in descs:
        d.wait()

    inv = jnp.float32(1.0 / CTX)

    def body(b, carry):
        base = b * CTX
        for c in range(EMB // L):
            acc = rows_v[base, pl.ds(c * L, L)]
            for t in range(1, CTX):
                acc = acc + rows_v[base + t, pl.ds(c * L, L)]
            pooled_v[b, pl.ds(c * L, L)] = acc * inv
        return carry

    lax.fori_loop(0, B_PER_W, body, 0)
    pltpu.sync_copy(pooled_v, out_hbm.at[pl.ds(wid * B_PER_W, B_PER_W)])


def _pool(emb_table, idx3):
    mesh = plsc.VectorSubcoreMesh(
        core_axis_name="c", subcore_axis_name="s",
        num_cores=NC, num_subcores=NS,
    )
    return pl.kernel(
        _pool_body,
        out_type=jax.ShapeDtypeStruct((BATCH, EMB), jnp.float32),
        mesh=mesh,
        compiler_params=pltpu.CompilerParams(use_tc_tiling_on_sc=False),
        scratch_types=[
            pltpu.VMEM((NCHUNK, GCHUNK), jnp.int32),
            pltpu.VMEM((IDX_PER_W, EMB), jnp.float32),
            pltpu.VMEM((B_PER_W, EMB), jnp.float32),
            pltpu.SemaphoreType.DMA,
        ],
    )(emb_table, idx3)


N_BLK = 1024
NGRID = (VOCAB + N_BLK - 1) // N_BLK


def _matmul_body(x_ref, w_ref, o_ref):
    o_ref[...] = lax.dot_general(
        x_ref[...], w_ref[...], (((1,), (1,)), ((), ())),
        preferred_element_type=jnp.float32,
    )


def _project(pooled, W):
    return pl.pallas_call(
        _matmul_body,
        grid=(NGRID,),
        in_specs=[
            pl.BlockSpec((BATCH, EMB), lambda i: (0, 0)),
            pl.BlockSpec((N_BLK, EMB), lambda i: (i, 0)),
        ],
        out_specs=pl.BlockSpec((BATCH, N_BLK), lambda i: (0, i)),
        out_shape=jax.ShapeDtypeStruct((BATCH, VOCAB), jnp.float32),
    )(pooled, W)


def kernel(inpt, emb_table, W):
    idx3 = inpt.astype(jnp.int32).reshape(NW, NCHUNK, GCHUNK)
    pooled = _pool(emb_table, idx3)
    return _project(pooled, W)


# speedup vs baseline: 2.8474x; 2.8474x over previous
import jax
import jax.numpy as jnp
from jax import lax
from jax.experimental import pallas as pl
from jax.experimental.pallas import tpu as pltpu
from jax.experimental.pallas import tpu_sc as plsc

VOCAB = 100000
EMB = 64
BATCH = 1024
CTX = 20

NC = 2
NS = 16
L = 16
NW = NC * NS
B_PER_W = BATCH // NW
IDX_PER_W = B_PER_W * CTX
GCHUNK = 128
NCHUNK = IDX_PER_W // GCHUNK


def _pool_body(table_hbm, idx_hbm, out_hbm, idx_v, rows_v, pooled_v, sem):
    wid = lax.axis_index("s") * NC + lax.axis_index("c")
    pltpu.sync_copy(idx_hbm.at[wid], idx_v)
    descs = [
        pltpu.async_copy(
            table_hbm.at[idx_v.at[j]],
            rows_v.at[pl.ds(j * GCHUNK, GCHUNK)],
            sem,
        )
        for j in range(NCHUNK)
    ]
    for d in descs:
        d.wait()

    inv = jnp.float32(1.0 / CTX)

    def body(b, carry):
        base = b * CTX
        for c in range(EMB // L):
            acc = rows_v[base, pl.ds(c * L, L)]
            for t in range(1, CTX):
                acc = acc + rows_v[base + t, pl.ds(c * L, L)]
            pooled_v[b, pl.ds(c * L, L)] = acc * inv
        return carry

    lax.fori_loop(0, B_PER_W, body, 0)
    pltpu.sync_copy(pooled_v, out_hbm.at[pl.ds(wid * B_PER_W, B_PER_W)])


def _pool(emb_table, idx3):
    mesh = plsc.VectorSubcoreMesh(
        core_axis_name="c", subcore_axis_name="s",
        num_cores=NC, num_subcores=NS,
    )
    return pl.kernel(
        _pool_body,
        out_type=jax.ShapeDtypeStruct((BATCH, EMB), jnp.float32),
        mesh=mesh,
        compiler_params=pltpu.CompilerParams(use_tc_tiling_on_sc=False),
        scratch_types=[
            pltpu.VMEM((NCHUNK, GCHUNK), jnp.int32),
            pltpu.VMEM((IDX_PER_W, EMB), jnp.float32),
            pltpu.VMEM((B_PER_W, EMB), jnp.float32),
            pltpu.SemaphoreType.DMA,
        ],
    )(emb_table, idx3)


M_BLK = 2048
NGRID = (VOCAB + M_BLK - 1) // M_BLK


def _matmul_body(wt_ref, xt_ref, o_ref):
    o_ref[...] = lax.dot_general(
        wt_ref[...], xt_ref[...], (((0,), (0,)), ((), ())),
        preferred_element_type=jnp.float32,
    )


def _project(pooled, W):
    out_t = pl.pallas_call(
        _matmul_body,
        grid=(NGRID,),
        in_specs=[
            pl.BlockSpec((EMB, M_BLK), lambda i: (0, i)),
            pl.BlockSpec((EMB, BATCH), lambda i: (0, 0)),
        ],
        out_specs=pl.BlockSpec((M_BLK, BATCH), lambda i: (i, 0)),
        out_shape=jax.ShapeDtypeStruct((VOCAB, BATCH), jnp.float32),
    )(W.T, pooled.T)
    return out_t.T


def kernel(inpt, emb_table, W):
    idx3 = inpt.astype(jnp.int32).reshape(NW, NCHUNK, GCHUNK)
    pooled = _pool(emb_table, idx3)
    return _project(pooled, W)
